# baseline (speedup 1.0000x reference)
import functools
import jax
import jax.numpy as jnp
from jax.experimental import pallas as pl
from jax.experimental.pallas import tpu as pltpu

_NKS_ALPHA = 0.6


def _image_kernel(num_cls, tile, ws, stride, x_ref, wt_ref, b_ref, s_ref,
                  cls_ref, loc_ref, fs_ref):
    rows = tile // ws
    ext = tile + 2 * ws
    lanes = ws * stride
    neg_inf = jnp.float32(-jnp.inf)

    s_mat = s_ref[...]
    for c in range(3):
        q = x_ref[0, c]
        for sy in range(4):
            g = jnp.dot(q[:, sy * lanes:(sy + 1) * lanes], s_mat,
                        preferred_element_type=jnp.float32)
            for sx in range(4):
                fs_ref[c * 16 + sy * 4 + sx] = g[:, sx * ws:sx * ws + ws]

    fx = fs_ref[...].reshape(48, tile)
    pred = jnp.dot(wt_ref[...], fx,
                   preferred_element_type=jnp.float32) + b_ref[...]

    cls_logits = pred[:num_cls, :]
    loc_raw = pred[num_cls:num_cls + 4, :]
    nks_logit = pred[num_cls + 4:num_cls + 5, :]

    gpos = jax.lax.broadcasted_iota(jnp.int32, (1, tile), 1)
    xc = ((gpos % ws) * stride + stride // 2).astype(jnp.float32)
    yc = ((gpos // ws) * stride + stride // 2).astype(jnp.float32)
    loc_row = jax.lax.broadcasted_iota(jnp.int32, (4, 1), 0)
    loc_sign = jnp.where(loc_row < 2, -1.0, 1.0)
    pix = jnp.where(loc_row % 2 == 0, xc, yc)
    loc_ref[0] = jnp.maximum(loc_raw, 0.0) * loc_sign + pix

    nks = jax.nn.sigmoid(nks_logit)
    nks = jax.nn.sigmoid(2.0 * nks - 1.0)
    exponent = (2.0 - nks) * _NKS_ALPHA + 1e-14
    log_p = -jax.nn.softplus(-cls_logits)
    cls_pred = jnp.exp(exponent * log_p)

    ninf_col = jnp.full((num_cls, ws), neg_inf, jnp.float32)
    cp = jnp.concatenate([ninf_col, cls_pred, ninf_col], axis=1)
    left = pltpu.roll(cp, 1, axis=1)[:, ws:ws + tile]
    right = pltpu.roll(cp, ext - 1, axis=1)[:, ws:ws + tile]
    hmax = jnp.maximum(jnp.maximum(left, cls_pred), right)

    pad = jnp.full((8, tile), neg_inf, jnp.float32)
    p = jnp.concatenate([hmax, pad], axis=0)
    up = pltpu.roll(p, 1, axis=0)[:num_cls, :]
    down = pltpu.roll(p, num_cls + 7, axis=0)[:num_cls, :]
    vmax = jnp.maximum(jnp.maximum(up, hmax), down)

    cls_ref[0] = jnp.where(vmax == cls_pred, cls_pred, 0.0)


@functools.partial(jax.jit, static_argnums=(4, 5))
def _detector(x, w, b, pixel_location, num_cls, stride):
    del pixel_location
    B, Cin, H, W = x.shape
    hs, ws = H // stride, W // stride
    HW = hs * ws
    K = Cin * stride * stride
    Ctot = num_cls + 5
    Cpad = ((Ctot + 7) // 8) * 8

    w_t = jnp.zeros((Cpad, K), jnp.float32).at[:Ctot].set(
        w.T.astype(jnp.float32))
    b_col = jnp.zeros((Cpad, 1), jnp.float32).at[:Ctot].set(
        b.reshape(Ctot, 1).astype(jnp.float32))

    xq = x.astype(jnp.float32).reshape(B, Cin, hs, stride * W)

    wcol = jnp.arange(W)
    sel = ((wcol[:, None] % stride) * ws + wcol[:, None] // stride
           == wcol[None, :]).astype(jnp.float32)

    body = functools.partial(_image_kernel, num_cls, HW, ws, stride)
    cls_out, loc_out = pl.pallas_call(
        body,
        out_shape=(jax.ShapeDtypeStruct((B, num_cls, HW), jnp.float32),
                   jax.ShapeDtypeStruct((B, 4, HW), jnp.float32)),
        grid=(B,),
        in_specs=[
            pl.BlockSpec((1, Cin, hs, stride * W), lambda i: (i, 0, 0, 0)),
            pl.BlockSpec((Cpad, K), lambda i: (0, 0)),
            pl.BlockSpec((Cpad, 1), lambda i: (0, 0)),
            pl.BlockSpec((W, W), lambda i: (0, 0)),
        ],
        out_specs=(pl.BlockSpec((1, num_cls, HW), lambda i: (i, 0, 0)),
                   pl.BlockSpec((1, 4, HW), lambda i: (i, 0, 0))),
        scratch_shapes=[pltpu.VMEM((K, hs, ws), jnp.float32)],
        compiler_params=pltpu.CompilerParams(
            dimension_semantics=("parallel",),
            vmem_limit_bytes=64 * 1024 * 1024),
    )(xq, w_t, b_col, sel)
    return cls_out, loc_out


def kernel(x, w, b, pixel_location):
    return _detector(x, w, b, pixel_location, 80, 4)
